# baseline (speedup 1.0000x reference)
import functools

import jax
import jax.numpy as jnp
from jax.experimental import pallas as pl
from jax.experimental.pallas import tpu as pltpu

_TOKENS = 32768
_D_IN = 4096
_H1 = 512
_H2 = 128
_E = 64
_K = 8
_TILE = 512


def _router_kernel(x_ref, w1_ref, b1_ref, w2_ref, b2_ref, w3_ref, b3_ref,
                   tkw_ref, tki_ref, aw_ref, logits_ref):
    logits = logits_ref[...]
    e = jnp.exp(logits)
    aw_ref[...] = e / jnp.sum(e, axis=-1, keepdims=True)

    lane = jax.lax.broadcasted_iota(
        jnp.int32, (_TILE, _E), 1).astype(jnp.float32)
    w = logits
    vals = []
    for _ in range(_K):
        v = jnp.max(w, axis=-1, keepdims=True)
        vals.append(v)
        w = jnp.where(w == v, -jnp.inf, w)
    idxs = [jnp.min(jnp.where(logits == v, lane, float(_E)),
                    axis=-1, keepdims=True)
            for v in vals]
    tv = jnp.exp(jnp.concatenate(vals, axis=-1))
    ti = jnp.concatenate(idxs, axis=-1).astype(jnp.int32)
    tkw_ref[...] = tv / jnp.sum(tv, axis=-1, keepdims=True)
    tki_ref[...] = ti

    x = x_ref[...]
    _KC = 512
    h = jnp.dot(x[:, :_KC], w1_ref[:_KC, :],
                preferred_element_type=jnp.float32)
    for c in range(1, _D_IN // _KC):
        h = h + jnp.dot(x[:, c * _KC:(c + 1) * _KC],
                        w1_ref[c * _KC:(c + 1) * _KC, :],
                        preferred_element_type=jnp.float32)
    h = jnp.maximum(h + b1_ref[...], 0.0)
    h = jnp.dot(h, w2_ref[...], preferred_element_type=jnp.float32)
    h = jnp.maximum(h + b2_ref[...], 0.0)
    logits_ref[...] = (jnp.dot(h, w3_ref[...], preferred_element_type=jnp.float32)
                       + b3_ref[...])


@functools.partial(jax.jit, static_argnames=())
def kernel(x, W1, b1, W2, b2, W3, b3):
    tokens = x.shape[0]
    ntiles = tokens // _TILE
    grid = (ntiles + 1,)
    out_shapes = (
        jax.ShapeDtypeStruct((tokens, _K), jnp.float32),
        jax.ShapeDtypeStruct((tokens, _K), jnp.int32),
        jax.ShapeDtypeStruct((tokens, _E), jnp.float32),
    )
    last = ntiles - 1
    x_map = lambda i: (jnp.minimum(i, last), 0)
    o_map = lambda i: (jnp.maximum(i - 1, 0), 0)
    const_spec = lambda shape: pl.BlockSpec(shape, lambda i: (0, 0))
    tkw, tki, aw = pl.pallas_call(
        _router_kernel,
        grid=grid,
        in_specs=[
            pl.BlockSpec((_TILE, _D_IN), x_map),
            const_spec((_D_IN, _H1)),
            const_spec((1, _H1)),
            const_spec((_H1, _H2)),
            const_spec((1, _H2)),
            const_spec((_H2, _E)),
            const_spec((1, _E)),
        ],
        out_specs=(
            pl.BlockSpec((_TILE, _K), o_map),
            pl.BlockSpec((_TILE, _K), o_map),
            pl.BlockSpec((_TILE, _E), o_map),
        ),
        out_shape=out_shapes,
        scratch_shapes=[pltpu.VMEM((_TILE, _E), jnp.float32)],
        compiler_params=pltpu.CompilerParams(
            dimension_semantics=("arbitrary",),
        ),
    )(x, W1, b1.reshape(1, _H1), W2, b2.reshape(1, _H2), W3,
      b3.reshape(1, _E))
    return (tkw, tki, aw)


# speedup vs baseline: 1.0964x; 1.0964x over previous
import functools

import jax
import jax.numpy as jnp
from jax.experimental import pallas as pl
from jax.experimental.pallas import tpu as pltpu

_TOKENS = 32768
_D_IN = 4096
_H1 = 512
_H2 = 128
_E = 64
_K = 8
_TILE = 1024


def _router_kernel(x_ref, w1_ref, b1_ref, w2_ref, b2_ref, w3_ref, b3_ref,
                   tkw_ref, tki_ref, aw_ref, logits_ref):
    logits = logits_ref[...]
    e = jnp.exp(logits)
    aw_ref[...] = e / jnp.sum(e, axis=-1, keepdims=True)

    lane = jax.lax.broadcasted_iota(
        jnp.int32, (_TILE, _E), 1).astype(jnp.float32)
    w = logits
    vals = []
    for _ in range(_K):
        v = jnp.max(w, axis=-1, keepdims=True)
        vals.append(v)
        w = jnp.where(w == v, -jnp.inf, w)
    idxs = [jnp.min(jnp.where(logits == v, lane, float(_E)),
                    axis=-1, keepdims=True)
            for v in vals]
    tv = jnp.exp(jnp.concatenate(vals, axis=-1))
    ti = jnp.concatenate(idxs, axis=-1).astype(jnp.int32)
    tkw_ref[...] = tv / jnp.sum(tv, axis=-1, keepdims=True)
    tki_ref[...] = ti

    x = x_ref[...]
    _KC = 512
    h = jnp.dot(x[:, :_KC], w1_ref[:_KC, :],
                preferred_element_type=jnp.float32)
    for c in range(1, _D_IN // _KC):
        h = h + jnp.dot(x[:, c * _KC:(c + 1) * _KC],
                        w1_ref[c * _KC:(c + 1) * _KC, :],
                        preferred_element_type=jnp.float32)
    h = jnp.maximum(h + b1_ref[...], 0.0)
    h = jnp.dot(h, w2_ref[...], preferred_element_type=jnp.float32)
    h = jnp.maximum(h + b2_ref[...], 0.0)
    logits_ref[...] = (jnp.dot(h, w3_ref[...], preferred_element_type=jnp.float32)
                       + b3_ref[...])


@functools.partial(jax.jit, static_argnames=())
def kernel(x, W1, b1, W2, b2, W3, b3):
    tokens = x.shape[0]
    ntiles = tokens // _TILE
    grid = (ntiles + 1,)
    out_shapes = (
        jax.ShapeDtypeStruct((tokens, _K), jnp.float32),
        jax.ShapeDtypeStruct((tokens, _K), jnp.int32),
        jax.ShapeDtypeStruct((tokens, _E), jnp.float32),
    )
    last = ntiles - 1
    x_map = lambda i: (jnp.minimum(i, last), 0)
    o_map = lambda i: (jnp.maximum(i - 1, 0), 0)
    const_spec = lambda shape: pl.BlockSpec(shape, lambda i: (0, 0))
    tkw, tki, aw = pl.pallas_call(
        _router_kernel,
        grid=grid,
        in_specs=[
            pl.BlockSpec((_TILE, _D_IN), x_map),
            const_spec((_D_IN, _H1)),
            const_spec((1, _H1)),
            const_spec((_H1, _H2)),
            const_spec((1, _H2)),
            const_spec((_H2, _E)),
            const_spec((1, _E)),
        ],
        out_specs=(
            pl.BlockSpec((_TILE, _K), o_map),
            pl.BlockSpec((_TILE, _K), o_map),
            pl.BlockSpec((_TILE, _E), o_map),
        ),
        out_shape=out_shapes,
        scratch_shapes=[pltpu.VMEM((_TILE, _E), jnp.float32)],
        compiler_params=pltpu.CompilerParams(
            dimension_semantics=("arbitrary",),
        ),
    )(x, W1, b1.reshape(1, _H1), W2, b2.reshape(1, _H2), W3,
      b3.reshape(1, _E))
    return (tkw, tki, aw)
